# Initial kernel scaffold; baseline (speedup 1.0000x reference)
import jax
import jax.numpy as jnp
from jax.experimental import pallas as pl


def kernel(inputs, embedding_weight):
    raise NotImplementedError("write your pallas kernel here")


# speedup vs baseline: 3.3252x; 3.3252x over previous
import functools

import jax
import jax.numpy as jnp
from jax import lax
from jax.experimental import pallas as pl
from jax.experimental.pallas import tpu as pltpu
from jax.experimental.pallas import tpu_sc as plsc

TM = 256
TK = 512
PAD = 128
TOPK = 3
CH = 128
CB = 16


SLOT = 8


def _dist_topk_body(x_ref, w_ref, d_ref, vals_ref, idx_ref, cnt_ref,
                    cv3, ci3, drow, ccnt):
    i = pl.program_id(0)
    j = pl.program_id(1)
    ni = pl.num_programs(0)
    nj = pl.num_programs(1)
    x = x_ref[...]
    w = w_ref[...]
    zn = jnp.sum(x * x, axis=1, keepdims=True)
    en = jnp.sum(w * w, axis=1)[None, :]
    acc = lax.dot_general(x, w, (((1,), (1,)), ((), ())),
                          preferred_element_type=jnp.float32)
    d = zn + en - 2.0 * acc
    d_ref[...] = d
    drow[j] = d

    colid = lax.broadcasted_iota(jnp.int32, (TM, TK), 1) + j * TK
    ms, ps = [], []
    dd = d
    for t in range(TOPK):
        m = jnp.min(dd, axis=1, keepdims=True)
        p = jnp.min(jnp.where(dd == m, colid, jnp.int32(2**30)),
                    axis=1, keepdims=True)
        ms.append(m)
        ps.append(p)
        if t < TOPK - 1:
            dd = jnp.where(colid == p, jnp.inf, dd)
    li = lax.broadcasted_iota(jnp.int32, (TM, SLOT), 1)
    cv3[j] = jnp.where(li == 0, ms[0],
                       jnp.where(li == 1, ms[1],
                                 jnp.where(li == 2, ms[2], jnp.inf)))
    ci3[j] = jnp.where(li == 0, ps[0],
                       jnp.where(li == 1, ps[1],
                                 jnp.where(li == 2, ps[2], 0)))

    @pl.when(j == nj - 1)
    def _out():
        cv = jnp.concatenate([cv3[t] for t in range(NJ)], axis=1)
        ci = jnp.concatenate([ci3[t] for t in range(NJ)], axis=1)
        ms2, ps2 = [], []
        for t in range(TOPK):
            m = jnp.min(cv, axis=1, keepdims=True)
            p = jnp.min(jnp.where(cv == m, ci, jnp.int32(2**30)),
                        axis=1, keepdims=True)
            ms2.append(m)
            ps2.append(p)
            if t < TOPK - 1:
                cv = jnp.where(ci == p, jnp.inf, cv)
        lo = lax.broadcasted_iota(jnp.int32, (TM, PAD), 1)
        vals_ref[...] = jnp.where(
            lo == 0, ms2[0],
            jnp.where(lo == 1, ms2[1],
                      jnp.where(lo == 2, ms2[2], jnp.inf)))
        idx_ref[...] = jnp.where(
            lo == 0, ps2[0],
            jnp.where(lo == 1, ps2[1],
                      jnp.where(lo == 2, ps2[2], 0)))

        thr = ms2[2]
        ones8 = jnp.ones((8, TM), jnp.float32)
        for t in range(NJ):
            indf = jnp.where(drow[t] <= thr, 1.0, 0.0)
            pc8 = lax.dot_general(ones8, indf, (((1,), (0,)), ((), ())),
                                  preferred_element_type=jnp.float32)
            ccnt[t] = jnp.where(i == 0, pc8, ccnt[t] + pc8)

        @pl.when(i == ni - 1)
        def _cnt_out():
            for t in range(NJ):
                cnt_ref[t] = ccnt[t]


NJ = 16


def _dist_topk(flat, w):
    n, c = flat.shape
    k = w.shape[0]
    nj = k // TK
    return pl.pallas_call(
        _dist_topk_body,
        grid=(n // TM, nj),
        in_specs=[pl.BlockSpec((TM, c), lambda i, j: (i, 0)),
                  pl.BlockSpec((TK, c), lambda i, j: (j, 0))],
        out_specs=[pl.BlockSpec((TM, TK), lambda i, j: (i, j)),
                   pl.BlockSpec((TM, PAD), lambda i, j: (i, 0)),
                   pl.BlockSpec((TM, PAD), lambda i, j: (i, 0)),
                   pl.BlockSpec((nj, 8, TK), lambda i, j: (0, 0, 0))],
        out_shape=[jax.ShapeDtypeStruct((n, k), jnp.float32),
                   jax.ShapeDtypeStruct((n, PAD), jnp.float32),
                   jax.ShapeDtypeStruct((n, PAD), jnp.int32),
                   jax.ShapeDtypeStruct((nj, 8, TK), jnp.float32)],
        scratch_shapes=[pltpu.VMEM((nj, TM, SLOT), jnp.float32),
                        pltpu.VMEM((nj, TM, SLOT), jnp.int32),
                        pltpu.VMEM((nj, TM, TK), jnp.float32),
                        pltpu.VMEM((nj, 8, TK), jnp.float32)],
        compiler_params=pltpu.CompilerParams(
            dimension_semantics=("arbitrary", "arbitrary")),
    )(flat, w)


def _make_sc_gather(k, c, nt):
    info = plsc.get_sparse_core_info()
    ncores, nsub = info.num_cores, info.num_subcores
    nw = ncores * nsub
    bpw = nt // nw
    nch = bpw // CH
    mesh = plsc.VectorSubcoreMesh(core_axis_name="c", subcore_axis_name="s")

    @functools.partial(
        pl.kernel, mesh=mesh,
        out_type=[jax.ShapeDtypeStruct((nt, c), jnp.float32)],
        scratch_types=[
            pltpu.VMEM((nch, CH), jnp.int32),
            pltpu.VMEM((CH, c), jnp.float32),
            pltpu.VMEM((CH, c), jnp.float32),
            pltpu.SemaphoreType.DMA,
            pltpu.SemaphoreType.DMA,
        ],
    )
    def sc_body(table_hbm, idx_hbm, out_hbm,
                idx2d, rows_a, rows_b, sem_a, sem_b):
        cid = lax.axis_index("c")
        sid = lax.axis_index("s")
        wid = sid * ncores + cid
        pltpu.sync_copy(idx_hbm.at[wid], idx2d)
        cur = pltpu.async_copy(table_hbm.at[idx2d.at[0]], rows_a, sem_a)
        for ch in range(nch):
            nxt = None
            if ch + 1 < nch:
                nxt = pltpu.async_copy(table_hbm.at[idx2d.at[ch + 1]],
                                       (rows_a, rows_b)[(ch + 1) % 2],
                                       (sem_a, sem_b)[(ch + 1) % 2])
            cur.wait()
            pltpu.sync_copy((rows_a, rows_b)[ch % 2],
                            out_hbm.at[pl.ds(wid * bpw + ch * CH, CH)])
            cur = nxt

    return sc_body, nw, nch


def _finalize_body(cnt_ref, vals_ref, loss_ref, perp_ref, avg_ref):
    n, pad = vals_ref.shape
    counts = cnt_ref[...][:, 0, :]
    avg = counts * (1.0 / n)
    avg_ref[...] = avg
    ent = jnp.sum(avg * jnp.log(avg + 1e-10))
    perp_ref[...] = jnp.exp(-ent).reshape(1, 1)
    v = vals_ref[...]
    li = lax.broadcasted_iota(jnp.int32, (n, pad), 1)
    s = jnp.sum(jnp.where(li < TOPK, v, 0.0))
    c = 256
    loss_ref[...] = (0.25 * s / (n * TOPK * c)).reshape(1, 1)


def _finalize(cnts, vals):
    nj, _, tk = cnts.shape
    return pl.pallas_call(
        _finalize_body,
        out_shape=[jax.ShapeDtypeStruct((1, 1), jnp.float32),
                   jax.ShapeDtypeStruct((1, 1), jnp.float32),
                   jax.ShapeDtypeStruct((nj, tk), jnp.float32)],
    )(cnts, vals)


def kernel(inputs, embedding_weight):
    b, t, c = inputs.shape
    k = embedding_weight.shape[0]
    n = b * t
    nt = n * TOPK
    flat = inputs.reshape(n, c)
    distances, vals8, idx8, cnts = _dist_topk(flat, embedding_weight)
    idx_top = idx8[:, :TOPK]
    sc, nw, nch = _make_sc_gather(k, c, nt)
    idx3d = idx_top.reshape(nw, nch, CH)
    quant_flat = sc(embedding_weight, idx3d)[0]
    loss, perp, avg = _finalize(cnts, vals8)
    return (loss.reshape(()),
            quant_flat.reshape(b, t, TOPK, c),
            perp.reshape(()),
            avg.reshape(k),
            idx_top.reshape(b, t, TOPK),
            distances.reshape(b, t, k))


# speedup vs baseline: 4.7658x; 1.4332x over previous
import functools

import jax
import jax.numpy as jnp
from jax import lax
from jax.experimental import pallas as pl
from jax.experimental.pallas import tpu as pltpu
from jax.experimental.pallas import tpu_sc as plsc

TM = 256
TK = 512
PAD = 128
TOPK = 3
CH = 128
CB = 16


SLOT = 8


def _dist_topk_body(x_ref, w_ref, d_ref, vals_ref, idx_ref, cnt_ref,
                    cv3, ci3, drow, ccnt):
    i = pl.program_id(0)
    j = pl.program_id(1)
    ni = pl.num_programs(0)
    nj = pl.num_programs(1)
    x = x_ref[...]
    w = w_ref[pl.ds(j * TK, TK), :]
    zn = jnp.sum(x * x, axis=1, keepdims=True)
    en = jnp.sum(w * w, axis=1)[None, :]
    acc = lax.dot_general(x, w, (((1,), (1,)), ((), ())),
                          preferred_element_type=jnp.float32)
    d = zn + en - 2.0 * acc
    d_ref[...] = d
    drow[j] = d

    colid = lax.broadcasted_iota(jnp.int32, (TM, TK), 1) + j * TK
    ms, ps = [], []
    dd = d
    for t in range(TOPK):
        m = jnp.min(dd, axis=1, keepdims=True)
        p = jnp.min(jnp.where(dd == m, colid, jnp.int32(2**30)),
                    axis=1, keepdims=True)
        ms.append(m)
        ps.append(p)
        if t < TOPK - 1:
            dd = jnp.where(colid == p, jnp.inf, dd)
    li = lax.broadcasted_iota(jnp.int32, (TM, SLOT), 1)
    cv3[j] = jnp.where(li == 0, ms[0],
                       jnp.where(li == 1, ms[1],
                                 jnp.where(li == 2, ms[2], jnp.inf)))
    ci3[j] = jnp.where(li == 0, ps[0],
                       jnp.where(li == 1, ps[1],
                                 jnp.where(li == 2, ps[2], 0)))

    @pl.when(j == nj - 1)
    def _out():
        cv = jnp.concatenate([cv3[t] for t in range(NJ)], axis=1)
        ci = jnp.concatenate([ci3[t] for t in range(NJ)], axis=1)
        ms2, ps2 = [], []
        for t in range(TOPK):
            m = jnp.min(cv, axis=1, keepdims=True)
            p = jnp.min(jnp.where(cv == m, ci, jnp.int32(2**30)),
                        axis=1, keepdims=True)
            ms2.append(m)
            ps2.append(p)
            if t < TOPK - 1:
                cv = jnp.where(ci == p, jnp.inf, cv)
        lo = lax.broadcasted_iota(jnp.int32, (TM, PAD), 1)
        vals_ref[...] = jnp.where(
            lo == 0, ms2[0],
            jnp.where(lo == 1, ms2[1],
                      jnp.where(lo == 2, ms2[2], jnp.inf)))
        idx_ref[...] = jnp.where(
            lo == 0, ps2[0],
            jnp.where(lo == 1, ps2[1],
                      jnp.where(lo == 2, ps2[2], 0)))

        thr = ms2[2]
        ones8 = jnp.ones((8, TM), jnp.float32)
        for t in range(NJ):
            indf = jnp.where(drow[t] <= thr, 1.0, 0.0)
            pc8 = lax.dot_general(ones8, indf, (((1,), (0,)), ((), ())),
                                  preferred_element_type=jnp.float32)
            ccnt[t] = jnp.where(i == 0, pc8, ccnt[t] + pc8)

        @pl.when(i == ni - 1)
        def _cnt_out():
            for t in range(NJ):
                cnt_ref[t] = ccnt[t]


NJ = 16


def _dist_topk(flat, w):
    n, c = flat.shape
    k = w.shape[0]
    nj = k // TK
    return pl.pallas_call(
        _dist_topk_body,
        grid=(n // TM, nj),
        in_specs=[pl.BlockSpec((TM, c), lambda i, j: (i, 0)),
                  pl.BlockSpec((k, c), lambda i, j: (0, 0))],
        out_specs=[pl.BlockSpec((TM, TK), lambda i, j: (i, j)),
                   pl.BlockSpec((TM, PAD), lambda i, j: (i, 0)),
                   pl.BlockSpec((TM, PAD), lambda i, j: (i, 0)),
                   pl.BlockSpec((nj, 8, TK), lambda i, j: (0, 0, 0))],
        out_shape=[jax.ShapeDtypeStruct((n, k), jnp.float32),
                   jax.ShapeDtypeStruct((n, PAD), jnp.float32),
                   jax.ShapeDtypeStruct((n, PAD), jnp.int32),
                   jax.ShapeDtypeStruct((nj, 8, TK), jnp.float32)],
        scratch_shapes=[pltpu.VMEM((nj, TM, SLOT), jnp.float32),
                        pltpu.VMEM((nj, TM, SLOT), jnp.int32),
                        pltpu.VMEM((nj, TM, TK), jnp.float32),
                        pltpu.VMEM((nj, 8, TK), jnp.float32)],
        compiler_params=pltpu.CompilerParams(
            dimension_semantics=("arbitrary", "arbitrary")),
    )(flat, w)


def _make_sc_gather(k, c, nt):
    info = plsc.get_sparse_core_info()
    ncores, nsub = info.num_cores, info.num_subcores
    nw = ncores * nsub
    bpw = nt // nw
    nch = bpw // CH
    mesh = plsc.VectorSubcoreMesh(core_axis_name="c", subcore_axis_name="s")

    @functools.partial(
        pl.kernel, mesh=mesh,
        out_type=[jax.ShapeDtypeStruct((nt, c), jnp.float32)],
        scratch_types=[
            pltpu.VMEM((nch, CH), jnp.int32),
            pltpu.VMEM((CH, c), jnp.float32),
            pltpu.VMEM((CH, c), jnp.float32),
            pltpu.SemaphoreType.DMA,
            pltpu.SemaphoreType.DMA,
        ],
    )
    def sc_body(table_hbm, idx_hbm, out_hbm,
                idx2d, rows_a, rows_b, sem_a, sem_b):
        cid = lax.axis_index("c")
        sid = lax.axis_index("s")
        wid = sid * ncores + cid
        pltpu.sync_copy(idx_hbm.at[wid], idx2d)
        cur = pltpu.async_copy(table_hbm.at[idx2d.at[0]], rows_a, sem_a)
        for ch in range(nch):
            nxt = None
            if ch + 1 < nch:
                nxt = pltpu.async_copy(table_hbm.at[idx2d.at[ch + 1]],
                                       (rows_a, rows_b)[(ch + 1) % 2],
                                       (sem_a, sem_b)[(ch + 1) % 2])
            cur.wait()
            pltpu.sync_copy((rows_a, rows_b)[ch % 2],
                            out_hbm.at[pl.ds(wid * bpw + ch * CH, CH)])
            cur = nxt

    return sc_body, nw, nch


def _finalize_body(cnt_ref, vals_ref, loss_ref, perp_ref, avg_ref):
    n, pad = vals_ref.shape
    counts = cnt_ref[...][:, 0, :]
    avg = counts * (1.0 / n)
    avg_ref[...] = avg
    ent = jnp.sum(avg * jnp.log(avg + 1e-10))
    perp_ref[...] = jnp.exp(-ent).reshape(1, 1)
    v = vals_ref[...]
    li = lax.broadcasted_iota(jnp.int32, (n, pad), 1)
    s = jnp.sum(jnp.where(li < TOPK, v, 0.0))
    c = 256
    loss_ref[...] = (0.25 * s / (n * TOPK * c)).reshape(1, 1)


def _finalize(cnts, vals):
    nj, _, tk = cnts.shape
    return pl.pallas_call(
        _finalize_body,
        out_shape=[jax.ShapeDtypeStruct((1, 1), jnp.float32),
                   jax.ShapeDtypeStruct((1, 1), jnp.float32),
                   jax.ShapeDtypeStruct((nj, tk), jnp.float32)],
    )(cnts, vals)


def kernel(inputs, embedding_weight):
    b, t, c = inputs.shape
    k = embedding_weight.shape[0]
    n = b * t
    nt = n * TOPK
    flat = inputs.reshape(n, c)
    distances, vals8, idx8, cnts = _dist_topk(flat, embedding_weight)
    idx_top = idx8[:, :TOPK]
    sc, nw, nch = _make_sc_gather(k, c, nt)
    idx3d = idx_top.reshape(nw, nch, CH)
    quant_flat = sc(embedding_weight, idx3d)[0]
    loss, perp, avg = _finalize(cnts, vals8)
    return (loss.reshape(()),
            quant_flat.reshape(b, t, TOPK, c),
            perp.reshape(()),
            avg.reshape(k),
            idx_top.reshape(b, t, TOPK),
            distances.reshape(b, t, k))


# speedup vs baseline: 4.7665x; 1.0002x over previous
import functools

import jax
import jax.numpy as jnp
from jax import lax
from jax.experimental import pallas as pl
from jax.experimental.pallas import tpu as pltpu
from jax.experimental.pallas import tpu_sc as plsc

TM = 512
TK = 512
PAD = 128
TOPK = 3
CH = 128
CB = 16


SLOT = 8


def _dist_topk_body(x_ref, w_ref, d_ref, vals_ref, idx_ref, cnt_ref,
                    cv3, ci3, drow, ccnt):
    i = pl.program_id(0)
    j = pl.program_id(1)
    ni = pl.num_programs(0)
    nj = pl.num_programs(1)
    x = x_ref[...]
    w = w_ref[pl.ds(j * TK, TK), :]
    zn = jnp.sum(x * x, axis=1, keepdims=True)
    en = jnp.sum(w * w, axis=1)[None, :]
    acc = lax.dot_general(x, w, (((1,), (1,)), ((), ())),
                          preferred_element_type=jnp.float32)
    d = zn + en - 2.0 * acc
    d_ref[...] = d
    drow[j] = d

    colid = lax.broadcasted_iota(jnp.int32, (TM, TK), 1) + j * TK
    ms, ps = [], []
    dd = d
    for t in range(TOPK):
        m = jnp.min(dd, axis=1, keepdims=True)
        p = jnp.min(jnp.where(dd == m, colid, jnp.int32(2**30)),
                    axis=1, keepdims=True)
        ms.append(m)
        ps.append(p)
        if t < TOPK - 1:
            dd = jnp.where(colid == p, jnp.inf, dd)
    li = lax.broadcasted_iota(jnp.int32, (TM, SLOT), 1)
    cv3[j] = jnp.where(li == 0, ms[0],
                       jnp.where(li == 1, ms[1],
                                 jnp.where(li == 2, ms[2], jnp.inf)))
    ci3[j] = jnp.where(li == 0, ps[0],
                       jnp.where(li == 1, ps[1],
                                 jnp.where(li == 2, ps[2], 0)))

    @pl.when(j == nj - 1)
    def _out():
        cv = jnp.concatenate([cv3[t] for t in range(NJ)], axis=1)
        ci = jnp.concatenate([ci3[t] for t in range(NJ)], axis=1)
        ms2, ps2 = [], []
        for t in range(TOPK):
            m = jnp.min(cv, axis=1, keepdims=True)
            p = jnp.min(jnp.where(cv == m, ci, jnp.int32(2**30)),
                        axis=1, keepdims=True)
            ms2.append(m)
            ps2.append(p)
            if t < TOPK - 1:
                cv = jnp.where(ci == p, jnp.inf, cv)
        lo = lax.broadcasted_iota(jnp.int32, (TM, PAD), 1)
        vals_ref[...] = jnp.where(
            lo == 0, ms2[0],
            jnp.where(lo == 1, ms2[1],
                      jnp.where(lo == 2, ms2[2], jnp.inf)))
        idx_ref[...] = jnp.where(
            lo == 0, ps2[0],
            jnp.where(lo == 1, ps2[1],
                      jnp.where(lo == 2, ps2[2], 0)))

        thr = ms2[2]
        ones8 = jnp.ones((8, TM), jnp.float32)
        for t in range(NJ):
            indf = jnp.where(drow[t] <= thr, 1.0, 0.0)
            pc8 = lax.dot_general(ones8, indf, (((1,), (0,)), ((), ())),
                                  preferred_element_type=jnp.float32)
            ccnt[t] = jnp.where(i == 0, pc8, ccnt[t] + pc8)

        @pl.when(i == ni - 1)
        def _cnt_out():
            for t in range(NJ):
                cnt_ref[t] = ccnt[t]


NJ = 16


def _dist_topk(flat, w):
    n, c = flat.shape
    k = w.shape[0]
    nj = k // TK
    return pl.pallas_call(
        _dist_topk_body,
        grid=(n // TM, nj),
        in_specs=[pl.BlockSpec((TM, c), lambda i, j: (i, 0)),
                  pl.BlockSpec((k, c), lambda i, j: (0, 0))],
        out_specs=[pl.BlockSpec((TM, TK), lambda i, j: (i, j)),
                   pl.BlockSpec((TM, PAD), lambda i, j: (i, 0)),
                   pl.BlockSpec((TM, PAD), lambda i, j: (i, 0)),
                   pl.BlockSpec((nj, 8, TK), lambda i, j: (0, 0, 0))],
        out_shape=[jax.ShapeDtypeStruct((n, k), jnp.float32),
                   jax.ShapeDtypeStruct((n, PAD), jnp.float32),
                   jax.ShapeDtypeStruct((n, PAD), jnp.int32),
                   jax.ShapeDtypeStruct((nj, 8, TK), jnp.float32)],
        scratch_shapes=[pltpu.VMEM((nj, TM, SLOT), jnp.float32),
                        pltpu.VMEM((nj, TM, SLOT), jnp.int32),
                        pltpu.VMEM((nj, TM, TK), jnp.float32),
                        pltpu.VMEM((nj, 8, TK), jnp.float32)],
        compiler_params=pltpu.CompilerParams(
            dimension_semantics=("arbitrary", "arbitrary")),
    )(flat, w)


def _make_sc_gather(k, c, nt):
    info = plsc.get_sparse_core_info()
    ncores, nsub = info.num_cores, info.num_subcores
    nw = ncores * nsub
    bpw = nt // nw
    nch = bpw // CH
    mesh = plsc.VectorSubcoreMesh(core_axis_name="c", subcore_axis_name="s")

    @functools.partial(
        pl.kernel, mesh=mesh,
        out_type=[jax.ShapeDtypeStruct((nt, c), jnp.float32)],
        scratch_types=[
            pltpu.VMEM((nch, CH), jnp.int32),
            pltpu.VMEM((CH, c), jnp.float32),
            pltpu.VMEM((CH, c), jnp.float32),
            pltpu.SemaphoreType.DMA,
            pltpu.SemaphoreType.DMA,
        ],
    )
    def sc_body(table_hbm, idx_hbm, out_hbm,
                idx2d, rows_a, rows_b, sem_a, sem_b):
        cid = lax.axis_index("c")
        sid = lax.axis_index("s")
        wid = sid * ncores + cid
        pltpu.sync_copy(idx_hbm.at[wid], idx2d)
        cur = pltpu.async_copy(table_hbm.at[idx2d.at[0]], rows_a, sem_a)
        for ch in range(nch):
            nxt = None
            if ch + 1 < nch:
                nxt = pltpu.async_copy(table_hbm.at[idx2d.at[ch + 1]],
                                       (rows_a, rows_b)[(ch + 1) % 2],
                                       (sem_a, sem_b)[(ch + 1) % 2])
            cur.wait()
            pltpu.sync_copy((rows_a, rows_b)[ch % 2],
                            out_hbm.at[pl.ds(wid * bpw + ch * CH, CH)])
            cur = nxt

    return sc_body, nw, nch


def _finalize_body(cnt_ref, vals_ref, loss_ref, perp_ref, avg_ref):
    n, pad = vals_ref.shape
    counts = cnt_ref[...][:, 0, :]
    avg = counts * (1.0 / n)
    avg_ref[...] = avg
    ent = jnp.sum(avg * jnp.log(avg + 1e-10))
    perp_ref[...] = jnp.exp(-ent).reshape(1, 1)
    v = vals_ref[...]
    li = lax.broadcasted_iota(jnp.int32, (n, pad), 1)
    s = jnp.sum(jnp.where(li < TOPK, v, 0.0))
    c = 256
    loss_ref[...] = (0.25 * s / (n * TOPK * c)).reshape(1, 1)


def _finalize(cnts, vals):
    nj, _, tk = cnts.shape
    return pl.pallas_call(
        _finalize_body,
        out_shape=[jax.ShapeDtypeStruct((1, 1), jnp.float32),
                   jax.ShapeDtypeStruct((1, 1), jnp.float32),
                   jax.ShapeDtypeStruct((nj, tk), jnp.float32)],
    )(cnts, vals)


def kernel(inputs, embedding_weight):
    b, t, c = inputs.shape
    k = embedding_weight.shape[0]
    n = b * t
    nt = n * TOPK
    flat = inputs.reshape(n, c)
    distances, vals8, idx8, cnts = _dist_topk(flat, embedding_weight)
    idx_top = idx8[:, :TOPK]
    sc, nw, nch = _make_sc_gather(k, c, nt)
    idx3d = idx_top.reshape(nw, nch, CH)
    quant_flat = sc(embedding_weight, idx3d)[0]
    loss, perp, avg = _finalize(cnts, vals8)
    return (loss.reshape(()),
            quant_flat.reshape(b, t, TOPK, c),
            perp.reshape(()),
            avg.reshape(k),
            idx_top.reshape(b, t, TOPK),
            distances.reshape(b, t, k))
